# baseline (speedup 1.0000x reference)
import jax
import jax.numpy as jnp
from jax.experimental import pallas as pl


def _sage_kern(adj_ref, x_ref, xr_ref, w_ref, o_ref):
    a = adj_ref[0]
    xb = x_ref[0]
    tmp = jnp.dot(a, xb, preferred_element_type=jnp.float32)
    res = tmp + xr_ref[0]
    o_ref[0] = jnp.dot(res, w_ref[...], preferred_element_type=jnp.float32)


def kernel(x, adj_t, W):
    B, S, _ = adj_t.shape
    N, IN = x.shape
    OUT = W.shape[1]
    TM = 512
    xb = x.reshape(B, S, IN)

    out = pl.pallas_call(
        _sage_kern,
        grid=(B, S // TM),
        in_specs=[
            pl.BlockSpec((1, TM, S), lambda b, i: (b, i, 0)),
            pl.BlockSpec((1, S, IN), lambda b, i: (b, 0, 0)),
            pl.BlockSpec((1, TM, IN), lambda b, i: (b, i, 0)),
            pl.BlockSpec((IN, OUT), lambda b, i: (0, 0)),
        ],
        out_specs=pl.BlockSpec((1, TM, OUT), lambda b, i: (b, i, 0)),
        out_shape=jax.ShapeDtypeStruct((B, S, OUT), jnp.float32),
    )(adj_t, xb, xb, W)
    return out.reshape(N, OUT)


# speedup vs baseline: 1.0018x; 1.0018x over previous
import jax
import jax.numpy as jnp
from jax.experimental import pallas as pl
from jax.experimental.pallas import tpu as pltpu


def _sage_kern(adj_ref, x_ref, xr_ref, w_ref, o_ref):
    a = adj_ref[0]
    xb = x_ref[0]
    tmp = jnp.dot(a, xb, preferred_element_type=jnp.float32)
    res = tmp + xr_ref[0]
    o_ref[0] = jnp.dot(res, w_ref[...], preferred_element_type=jnp.float32)


def kernel(x, adj_t, W):
    B, S, _ = adj_t.shape
    N, IN = x.shape
    OUT = W.shape[1]
    TM = 512
    xb = x.reshape(B, S, IN)

    out = pl.pallas_call(
        _sage_kern,
        grid=(B, S // TM),
        in_specs=[
            pl.BlockSpec((1, TM, S), lambda b, i: (b, i, 0)),
            pl.BlockSpec((1, S, IN), lambda b, i: (b, 0, 0)),
            pl.BlockSpec((1, TM, IN), lambda b, i: (b, i, 0)),
            pl.BlockSpec((IN, OUT), lambda b, i: (0, 0)),
        ],
        out_specs=pl.BlockSpec((1, TM, OUT), lambda b, i: (b, i, 0)),
        out_shape=jax.ShapeDtypeStruct((B, S, OUT), jnp.float32),
        compiler_params=pltpu.CompilerParams(
            dimension_semantics=("parallel", "parallel"),
        ),
    )(adj_t, xb, xb, W)
    return out.reshape(N, OUT)
